# baseline (speedup 1.0000x reference)
import math

import jax
import jax.numpy as jnp
from jax import lax
from jax.experimental import pallas as pl
from jax.experimental.pallas import tpu as pltpu
from jax.experimental.pallas import tpu_sc as plsc

D_MODEL = 64
SCALE = math.sqrt(D_MODEL)

_NC = 2
_NS = 16
_NW = _NC * _NS

_CHUNK = 512
_IDX_PER_STREAM = 128


def _make_embed(B: int):
    assert B % (_NW * _CHUNK) == 0, B
    bpw = B // _NW
    nchunk = bpw // _CHUNK
    ngath = _CHUNK // _IDX_PER_STREAM

    mesh = plsc.VectorSubcoreMesh(core_axis_name="c", subcore_axis_name="s")

    def body(table_hbm, idx_hbm, out_hbm, idx_v, rows_v, sem):
        wid = lax.axis_index("s") * _NC + lax.axis_index("c")
        base = wid * bpw

        @pl.loop(0, nchunk)
        def _chunk(g):
            start = base + g * _CHUNK
            pltpu.sync_copy(idx_hbm.at[pl.ds(start, _CHUNK)], idx_v)
            descs = [
                pltpu.async_copy(
                    table_hbm.at[idx_v.at[pl.ds(j * _IDX_PER_STREAM,
                                                _IDX_PER_STREAM)]],
                    rows_v.at[pl.ds(j * _IDX_PER_STREAM, _IDX_PER_STREAM)],
                    sem,
                )
                for j in range(ngath)
            ]
            for d in descs:
                d.wait()

            @pl.loop(0, _CHUNK)
            def _scale(r):
                for j in range(D_MODEL // 16):
                    sl = pl.ds(j * 16, 16)
                    rows_v[r, sl] = rows_v[r, sl] * SCALE

            pltpu.sync_copy(rows_v, out_hbm.at[pl.ds(start, _CHUNK)])

    return pl.kernel(
        body,
        out_type=jax.ShapeDtypeStruct((B, D_MODEL), jnp.float32),
        mesh=mesh,
        scratch_types=[
            pltpu.VMEM((_CHUNK,), jnp.int32),
            pltpu.VMEM((_CHUNK, D_MODEL), jnp.float32),
            pltpu.SemaphoreType.DMA,
        ],
        compiler_params=pltpu.CompilerParams(use_tc_tiling_on_sc=False),
    )


def kernel(x, table):
    batch, hist = x.shape
    idx = x.reshape(-1).astype(jnp.int32)
    out = _make_embed(idx.shape[0])(table, idx)
    return out.reshape(batch, hist, D_MODEL)
